# baseline (speedup 1.0000x reference)
import jax
import jax.numpy as jnp
from jax.experimental import pallas as pl
from jax.experimental.pallas import tpu as pltpu

_GRID = 2
_CHUNK = 512
_NBUF = 4


def _relay_body(in_hbm, out_hbm, buf, in_sems, out_sems):
    rows = in_hbm.shape[0]
    per_core = rows // _GRID
    nchunks = per_core // _CHUNK
    base = pl.program_id(0) * per_core

    in_copies = [
        pltpu.make_async_copy(
            in_hbm.at[pl.ds(base + c * _CHUNK, _CHUNK), :],
            buf.at[c % _NBUF],
            in_sems.at[c % _NBUF],
        )
        for c in range(nchunks)
    ]
    out_copies = [
        pltpu.make_async_copy(
            buf.at[c % _NBUF],
            out_hbm.at[pl.ds(base + c * _CHUNK, _CHUNK), :],
            out_sems.at[c % _NBUF],
        )
        for c in range(nchunks)
    ]

    in_copies[0].start()
    for c in range(nchunks):
        in_copies[c].wait()
        out_copies[c].start()
        if c + 1 < nchunks:
            if c + 1 >= _NBUF:
                out_copies[c + 1 - _NBUF].wait()
            in_copies[c + 1].start()
    for c in range(max(0, nchunks - _NBUF), nchunks):
        out_copies[c].wait()


def kernel(x, embed_weight):
    del embed_weight
    b, s, d = x.shape
    rows = b * s
    x2 = x.reshape(rows, d)
    out = pl.pallas_call(
        _relay_body,
        out_shape=jax.ShapeDtypeStruct((rows, d), x.dtype),
        grid=(_GRID,),
        in_specs=[pl.BlockSpec(memory_space=pltpu.MemorySpace.HBM)],
        out_specs=pl.BlockSpec(memory_space=pltpu.MemorySpace.HBM),
        scratch_shapes=[
            pltpu.VMEM((_NBUF, _CHUNK, d), jnp.float32),
            pltpu.SemaphoreType.DMA((_NBUF,)),
            pltpu.SemaphoreType.DMA((_NBUF,)),
        ],
        compiler_params=pltpu.CompilerParams(
            dimension_semantics=("parallel",),
        ),
    )(x2)
    return out.reshape(b, s, d)


# speedup vs baseline: 1.9082x; 1.9082x over previous
import jax
import jax.numpy as jnp
from jax.experimental import pallas as pl
from jax.experimental.pallas import tpu as pltpu


def _copy_body(in_ref, out_ref):
    out_ref[...] = in_ref[...]


def kernel(x, embed_weight):
    del embed_weight
    b, s, d = x.shape
    rows = b * s
    x2 = x.reshape(rows, d)
    block_rows = 2048
    out = pl.pallas_call(
        _copy_body,
        out_shape=jax.ShapeDtypeStruct((rows, d), x.dtype),
        grid=(rows // block_rows,),
        in_specs=[pl.BlockSpec((block_rows, d), lambda i: (i, 0))],
        out_specs=pl.BlockSpec((block_rows, d), lambda i: (i, 0)),
        compiler_params=pltpu.CompilerParams(
            dimension_semantics=("parallel",),
        ),
    )(x2)
    return out.reshape(b, s, d)
